# Initial kernel scaffold; baseline (speedup 1.0000x reference)
import jax
import jax.numpy as jnp
from jax.experimental import pallas as pl


def kernel(x_litter, edge_index_rel1, edge_index_rel2, proj_W, proj_b, att_src_1, att_dst_1, att_src_2, att_dst_2, k_W, k_b, q, lin_W, lin_b):
    raise NotImplementedError("write your pallas kernel here")


# speedup vs baseline: 23.0094x; 23.0094x over previous
import functools

import jax
import jax.numpy as jnp
from jax import lax
from jax.experimental import pallas as pl
from jax.experimental.pallas import tpu as pltpu
from jax.experimental.pallas import tpu_sc as plsc

N = 50000
E = 400000
D_IN = 128
HID = 128
HEADS = 8
DIM = 16
OUT = 16

_BLK = 1000

NC = 2
NS = 16
NW = NC * NS

B1 = 128
BT = 64
EPW = 12800
NE_PAD = NW * EPW
NS_TAB = 53248
CH = 8192
NCH = 7
ACC_R = CH + 16
NROW = NCH * CH

_SC_PARAMS = pltpu.CompilerParams(use_tc_tiling_on_sc=False,
                                  needs_layout_passes=False)


def _mesh():
    return plsc.VectorSubcoreMesh(core_axis_name="c", subcore_axis_name="s",
                                  num_cores=NC, num_subcores=NS)


def _p1_body(asd_hbm, add_hbm, src_hbm, dst_hbm, zs_hbm,
             e_hbm, s_hbm,
             src_v, dst_v, sbuf, dbuf, ebuf, s_sh, sem1, sem2):
    c = lax.axis_index("c")
    s_ = lax.axis_index("s")
    wid = s_ * NC + c

    zrows = NS_TAB // NS
    pltpu.sync_copy(zs_hbm, s_sh.at[pl.ds(s_ * zrows, zrows)])
    plsc.subcore_barrier()

    def batch(b, carry):
        base = wid * EPW + b * B1
        pltpu.sync_copy(src_hbm.at[pl.ds(base, B1)], src_v)
        pltpu.sync_copy(dst_hbm.at[pl.ds(base, B1)], dst_v)
        cp1 = pltpu.async_copy(asd_hbm.at[src_v], sbuf, sem1)
        cp2 = pltpu.async_copy(add_hbm.at[dst_v], dbuf, sem2)
        cp1.wait()
        cp2.wait()

        def edge(i, carry2):
            a = sbuf[i] + dbuf[i]
            a = jnp.where(a > 0.0, a, 0.2 * a)
            ebuf[i] = jnp.exp(a)
            return carry2

        lax.fori_loop(0, B1, edge, 0)
        pltpu.sync_copy(ebuf, e_hbm.at[pl.ds(base, B1)])
        pltpu.sync_copy(ebuf, s_sh.at[dst_v], add=True)
        return carry

    lax.fori_loop(0, EPW // B1, batch, 0)
    plsc.subcore_barrier()
    pltpu.sync_copy(s_sh.at[pl.ds(s_ * zrows, zrows)],
                    s_hbm.at[c, pl.ds(s_ * zrows, zrows)])


def _p1(asd, add, src, dst, zs):
    f = pl.kernel(
        _p1_body,
        out_type=[
            jax.ShapeDtypeStruct((NE_PAD + BT, 16), jnp.float32),
            jax.ShapeDtypeStruct((NC, NS_TAB, 16), jnp.float32),
        ],
        mesh=_mesh(),
        compiler_params=_SC_PARAMS,
        scratch_types=[
            pltpu.VMEM((B1,), jnp.int32),
            pltpu.VMEM((B1,), jnp.int32),
            pltpu.VMEM((B1, 16), jnp.float32),
            pltpu.VMEM((B1, 16), jnp.float32),
            pltpu.VMEM((B1, 16), jnp.float32),
            pltpu.VMEM_SHARED((NS_TAB, 16), jnp.float32),
            pltpu.SemaphoreType.DMA,
            pltpu.SemaphoreType.DMA,
        ],
    )
    return f(asd, add, src, dst, zs)


def _p2_body(h_hbm, src_hbm, dst_hbm, e_hbm, za_hbm,
             acc_hbm,
             srcw, dstw, cidx, sb, db, eb, hbuf, ebuf, msg, acc_sh,
             sem1, sem2):
    c = lax.axis_index("c")
    s_ = lax.axis_index("s")
    wid = s_ * NC + c
    wbase = wid * EPW

    pltpu.sync_copy(src_hbm.at[pl.ds(wbase, EPW)], srcw.at[pl.ds(0, EPW)])
    pltpu.sync_copy(dst_hbm.at[pl.ds(wbase, EPW)], dstw.at[pl.ds(0, EPW)])
    for k in range(BT // 16):
        srcw[pl.ds(EPW + k * 16, 16)] = jnp.zeros((16,), jnp.int32)
        dstw[pl.ds(EPW + k * 16, 16)] = jnp.full((16,), 2 ** 30, jnp.int32)

    for ch in range(NCH):
        lo = ch * CH
        zrows = ACC_R // NS
        pltpu.sync_copy(za_hbm, acc_sh.at[pl.ds(s_ * zrows, zrows)])
        plsc.subcore_barrier()

        def cmp_body(j, cnt):
            dv = dstw[pl.ds(j * 16, 16)]
            lv = dv - lo
            m = (lv >= 0) & (lv < CH)
            mi = m.astype(jnp.int32)
            idxv = lax.iota(jnp.int32, 16) + j * 16
            pos = jnp.where(m, plsc.cumsum(mi) + (cnt - 1), EPW + BT - 1)
            plsc.store_scatter(cidx, [pos], idxv)
            return cnt + jnp.sum(mi)

        cnt = lax.fori_loop(0, EPW // 16, cmp_body, 0)
        for k in range(BT // 16):
            cidx[pl.ds(cnt + k * 16, 16)] = jnp.full((16,), EPW, jnp.int32)

        nb = (cnt + BT - 1) // BT

        def bat(g, carry):
            gb = g * BT
            for k in range(BT // 16):
                ids = cidx[pl.ds(gb + k * 16, 16)]
                sv = plsc.load_gather(srcw, [ids])
                dv = plsc.load_gather(dstw, [ids])
                lv = dv - lo
                ok = (lv >= 0) & (lv < CH)
                dl = jnp.where(ok, lv, CH)
                sb[pl.ds(k * 16, 16)] = sv
                db[pl.ds(k * 16, 16)] = dl
                eb[pl.ds(k * 16, 16)] = ids + wbase
            cph = pltpu.async_copy(h_hbm.at[sb], hbuf, sem1)
            cpe = pltpu.async_copy(e_hbm.at[eb], ebuf, sem2)
            cph.wait()
            cpe.wait()

            def edge(i, carry2):
                for hd in range(HEADS):
                    ev = plsc.load_gather(
                        ebuf, [jnp.full((16,), i, jnp.int32),
                               jnp.full((16,), hd, jnp.int32)])
                    hv = hbuf[i, pl.ds(hd * 16, 16)]
                    msg[i, pl.ds(hd * 16, 16)] = hv * ev
                return carry2

            lax.fori_loop(0, BT, edge, 0)
            pltpu.sync_copy(msg, acc_sh.at[db], add=True)
            return carry

        lax.fori_loop(0, nb, bat, 0)
        plsc.subcore_barrier()
        frows = CH // NS
        pltpu.sync_copy(acc_sh.at[pl.ds(s_ * frows, frows)],
                        acc_hbm.at[c, pl.ds(lo + s_ * frows, frows)])
        plsc.subcore_barrier()


def _p2(h, src, dst, e, za):
    f = pl.kernel(
        _p2_body,
        out_type=jax.ShapeDtypeStruct((NC, NROW, HID), jnp.float32),
        mesh=_mesh(),
        compiler_params=_SC_PARAMS,
        scratch_types=[
            pltpu.VMEM((EPW + BT,), jnp.int32),
            pltpu.VMEM((EPW + BT,), jnp.int32),
            pltpu.VMEM((EPW + BT,), jnp.int32),
            pltpu.VMEM((BT,), jnp.int32),
            pltpu.VMEM((BT,), jnp.int32),
            pltpu.VMEM((BT,), jnp.int32),
            pltpu.VMEM((BT, HID), jnp.float32),
            pltpu.VMEM((BT, 16), jnp.float32),
            pltpu.VMEM((BT, HID), jnp.float32),
            pltpu.VMEM_SHARED((ACC_R, HID), jnp.float32),
            pltpu.SemaphoreType.DMA,
            pltpu.SemaphoreType.DMA,
        ],
    )
    return f(h, src, dst, e, za)


def _pre_body(x_ref, w_ref, b_ref, a_ref, h_ref, al_ref):
    x = x_ref[...]
    h = jnp.dot(x, w_ref[...], preferred_element_type=jnp.float32) + b_ref[...]
    h_ref[...] = h
    al_ref[...] = jnp.dot(h, a_ref[...], preferred_element_type=jnp.float32)


def _pre(x, w, b, acat):
    return pl.pallas_call(
        _pre_body,
        grid=(N // _BLK,),
        in_specs=[
            pl.BlockSpec((_BLK, D_IN), lambda i: (i, 0)),
            pl.BlockSpec((D_IN, HID), lambda i: (0, 0)),
            pl.BlockSpec((1, HID), lambda i: (0, 0)),
            pl.BlockSpec((HID, 32), lambda i: (0, 0)),
        ],
        out_specs=[
            pl.BlockSpec((_BLK, HID), lambda i: (i, 0)),
            pl.BlockSpec((_BLK, 32), lambda i: (i, 0)),
        ],
        out_shape=[
            jax.ShapeDtypeStruct((N, HID), jnp.float32),
            jax.ShapeDtypeStruct((N, 32), jnp.float32),
        ],
    )(x, w, b.reshape(1, HID), acat)


def _post_body(acc1, acc2, s1, s2, kw_ref, kb_ref,
               o1_ref, o2_ref, t_ref):
    @pl.when(pl.program_id(0) == 0)
    def _():
        t_ref[...] = jnp.zeros_like(t_ref)

    row = lax.broadcasted_iota(jnp.int32, (16, HID), 0)
    col = lax.broadcasted_iota(jnp.int32, (16, HID), 1)
    expm = (col // DIM == row).astype(jnp.float32)

    def one(acc, s3):
        s = s3[0] + s3[1]
        r = 1.0 / (s + 1e-16)
        rexp = jnp.dot(r, expm, preferred_element_type=jnp.float32)
        return jax.nn.relu((acc[0] + acc[1]) * rexp)

    o1 = one(acc1, s1)
    o2 = one(acc2, s2)
    o1_ref[...] = o1
    o2_ref[...] = o2
    t1 = jnp.tanh(jnp.dot(o1, kw_ref[...],
                          preferred_element_type=jnp.float32) + kb_ref[...])
    t2 = jnp.tanh(jnp.dot(o2, kw_ref[...],
                          preferred_element_type=jnp.float32) + kb_ref[...])
    t_ref[0, :] += jnp.sum(t1, axis=0)
    t_ref[1, :] += jnp.sum(t2, axis=0)


def _post(acc1, acc2, s1, s2, k_W, k_b):
    bspec_a = pl.BlockSpec((NC, _BLK, HID), lambda i: (0, i, 0))
    bspec_s = pl.BlockSpec((NC, _BLK, 16), lambda i: (0, i, 0))
    return pl.pallas_call(
        _post_body,
        grid=(N // _BLK,),
        in_specs=[bspec_a, bspec_a, bspec_s, bspec_s,
                  pl.BlockSpec((HID, HID), lambda i: (0, 0)),
                  pl.BlockSpec((1, HID), lambda i: (0, 0))],
        out_specs=[pl.BlockSpec((_BLK, HID), lambda i: (i, 0)),
                   pl.BlockSpec((_BLK, HID), lambda i: (i, 0)),
                   pl.BlockSpec((2, HID), lambda i: (0, 0))],
        out_shape=[jax.ShapeDtypeStruct((N, HID), jnp.float32),
                   jax.ShapeDtypeStruct((N, HID), jnp.float32),
                   jax.ShapeDtypeStruct((2, HID), jnp.float32)],
    )(acc1, acc2, s1, s2, k_W, k_b.reshape(1, HID))


def _final_body(o1_ref, o2_ref, w_ref, b_ref, beta_ref, out_ref):
    mix = beta_ref[0, 0] * o1_ref[...] + beta_ref[0, 1] * o2_ref[...]
    out_ref[...] = jnp.dot(mix, w_ref[...],
                           preferred_element_type=jnp.float32) + b_ref[...]


def _final(o1, o2, lin_W, lin_b, beta):
    return pl.pallas_call(
        _final_body,
        grid=(N // _BLK,),
        in_specs=[
            pl.BlockSpec((_BLK, HID), lambda i: (i, 0)),
            pl.BlockSpec((_BLK, HID), lambda i: (i, 0)),
            pl.BlockSpec((HID, OUT), lambda i: (0, 0)),
            pl.BlockSpec((1, OUT), lambda i: (0, 0)),
            pl.BlockSpec(memory_space=pltpu.SMEM),
        ],
        out_specs=pl.BlockSpec((_BLK, OUT), lambda i: (i, 0)),
        out_shape=jax.ShapeDtypeStruct((N, OUT), jnp.float32),
    )(o1, o2, lin_W, lin_b.reshape(1, OUT), beta.reshape(1, 2))


def kernel(x_litter, edge_index_rel1, edge_index_rel2, proj_W, proj_b,
           att_src_1, att_dst_1, att_src_2, att_dst_2,
           k_W, k_b, q, lin_W, lin_b):
    def expand(att):
        eye = jnp.eye(HEADS, dtype=jnp.float32)
        return (att[:, :, None] * eye[:, None, :]).reshape(HID, HEADS)

    acat = jnp.concatenate([expand(att_src_1), expand(att_dst_1),
                            expand(att_src_2), expand(att_dst_2)], axis=1)

    h, al = _pre(x_litter, proj_W, proj_b, acat)

    def dup(a8):
        return jnp.pad(jnp.concatenate([a8, a8], axis=1),
                       ((0, NS_TAB - N), (0, 0)))

    asd1 = dup(al[:, 0:8])
    add1 = dup(al[:, 8:16])
    asd2 = dup(al[:, 16:24])
    add2 = dup(al[:, 24:32])

    def padded(ei):
        src = jnp.pad(ei[0], (0, NE_PAD - E))
        dst = jnp.pad(ei[1], (0, NE_PAD - E), constant_values=N)
        return src, dst

    src1, dst1 = padded(edge_index_rel1)
    src2, dst2 = padded(edge_index_rel2)

    zs = jnp.zeros((NS_TAB // NS, 16), jnp.float32)
    za = jnp.zeros((ACC_R // NS, HID), jnp.float32)

    e1, s1 = _p1(asd1, add1, src1, dst1, zs)
    e2, s2 = _p1(asd2, add2, src2, dst2, zs)
    acc1 = _p2(h, src1, dst1, e1, za)
    acc2 = _p2(h, src2, dst2, e2, za)

    o1, o2, tsum = _post(acc1, acc2, s1, s2, k_W, k_b)
    score = (q[None, :] * (tsum / N)).sum(-1)
    beta = jax.nn.softmax(score, axis=0)
    return _final(o1, o2, lin_W, lin_b, beta)


# speedup vs baseline: 28.8652x; 1.2545x over previous
import functools

import jax
import jax.numpy as jnp
from jax import lax
from jax.experimental import pallas as pl
from jax.experimental.pallas import tpu as pltpu
from jax.experimental.pallas import tpu_sc as plsc

N = 50000
E = 400000
D_IN = 128
HID = 128
HEADS = 8
DIM = 16
OUT = 16

_BLK = 1000

NC = 2
NS = 16
NW = NC * NS

B1 = 128
BT = 64
EPW = 12800
NE_PAD = NW * EPW
NS_TAB = 53248
CH = 6272
NCH = 8
ACC_R = CH + 16
NROW = NCH * CH

_SC_PARAMS = pltpu.CompilerParams(use_tc_tiling_on_sc=False,
                                  needs_layout_passes=False)


def _mesh():
    return plsc.VectorSubcoreMesh(core_axis_name="c", subcore_axis_name="s",
                                  num_cores=NC, num_subcores=NS)


def _p1_body(asd_hbm, add_hbm, src_hbm, dst_hbm, zs_hbm,
             e_hbm, s_hbm,
             src_v, dst_v, sbuf, dbuf, ebuf, s_sh, sem1, sem2):
    c = lax.axis_index("c")
    s_ = lax.axis_index("s")
    wid = s_ * NC + c

    zrows = NS_TAB // NS
    pltpu.sync_copy(zs_hbm, s_sh.at[pl.ds(s_ * zrows, zrows)])
    plsc.subcore_barrier()

    def batch(b, carry):
        base = wid * EPW + b * B1
        pltpu.sync_copy(src_hbm.at[pl.ds(base, B1)], src_v)
        pltpu.sync_copy(dst_hbm.at[pl.ds(base, B1)], dst_v)
        cp1 = pltpu.async_copy(asd_hbm.at[src_v], sbuf, sem1)
        cp2 = pltpu.async_copy(add_hbm.at[dst_v], dbuf, sem2)
        cp1.wait()
        cp2.wait()

        def edge(i, carry2):
            a = sbuf[i] + dbuf[i]
            a = jnp.where(a > 0.0, a, 0.2 * a)
            ebuf[i] = jnp.exp(a)
            return carry2

        lax.fori_loop(0, B1, edge, 0)
        pltpu.sync_copy(ebuf, e_hbm.at[pl.ds(base, B1)])
        pltpu.sync_copy(ebuf, s_sh.at[dst_v], add=True)
        return carry

    lax.fori_loop(0, EPW // B1, batch, 0)
    plsc.subcore_barrier()
    pltpu.sync_copy(s_sh.at[pl.ds(s_ * zrows, zrows)],
                    s_hbm.at[c, pl.ds(s_ * zrows, zrows)])


def _p1(asd, add, src, dst, zs):
    f = pl.kernel(
        _p1_body,
        out_type=[
            jax.ShapeDtypeStruct((NE_PAD + BT, 16), jnp.float32),
            jax.ShapeDtypeStruct((NC, NS_TAB, 16), jnp.float32),
        ],
        mesh=_mesh(),
        compiler_params=_SC_PARAMS,
        scratch_types=[
            pltpu.VMEM((B1,), jnp.int32),
            pltpu.VMEM((B1,), jnp.int32),
            pltpu.VMEM((B1, 16), jnp.float32),
            pltpu.VMEM((B1, 16), jnp.float32),
            pltpu.VMEM((B1, 16), jnp.float32),
            pltpu.VMEM_SHARED((NS_TAB, 16), jnp.float32),
            pltpu.SemaphoreType.DMA,
            pltpu.SemaphoreType.DMA,
        ],
    )
    return f(asd, add, src, dst, zs)


def _p2_body(h_hbm, src_hbm, dst_hbm, e_hbm, za_hbm,
             acc_hbm,
             srcw, dstw, cidx,
             sba, dba, eba, hbufa, ebufa, msga,
             sbb, dbb, ebb, hbufb, ebufb, msgb,
             acc_sh, semha, semea, semhb, semeb):
    c = lax.axis_index("c")
    s_ = lax.axis_index("s")
    wid = s_ * NC + c
    wbase = wid * EPW

    pltpu.sync_copy(src_hbm.at[pl.ds(wbase, EPW)], srcw.at[pl.ds(0, EPW)])
    pltpu.sync_copy(dst_hbm.at[pl.ds(wbase, EPW)], dstw.at[pl.ds(0, EPW)])
    for k in range(2 * BT // 16):
        srcw[pl.ds(EPW + k * 16, 16)] = jnp.zeros((16,), jnp.int32)
        dstw[pl.ds(EPW + k * 16, 16)] = jnp.full((16,), 2 ** 30, jnp.int32)

    def scale(hbuf, ebuf, msg):
        def edge(i, carry2):
            erow = ebuf[i]
            for hd in range(HEADS):
                ev = lax.gather(
                    erow, jnp.full((16, 1), hd, jnp.int32),
                    dimension_numbers=lax.GatherDimensionNumbers(
                        offset_dims=(), collapsed_slice_dims=(0,),
                        start_index_map=(0,)),
                    slice_sizes=(1,),
                    mode=lax.GatherScatterMode.PROMISE_IN_BOUNDS)
                hv = hbuf[i, pl.ds(hd * 16, 16)]
                msg[i, pl.ds(hd * 16, 16)] = hv * ev
            return carry2

        lax.fori_loop(0, BT, edge, 0)

    for ch in range(NCH):
        lo = ch * CH
        zrows = ACC_R // NS
        pltpu.sync_copy(za_hbm, acc_sh.at[pl.ds(s_ * zrows, zrows)])
        plsc.subcore_barrier()

        def cmp_body(j, cnt):
            dv = dstw[pl.ds(j * 16, 16)]
            lv = dv - lo
            m = (lv >= 0) & (lv < CH)
            mi = m.astype(jnp.int32)
            idxv = lax.iota(jnp.int32, 16) + j * 16
            pos = jnp.where(m, plsc.cumsum(mi) + (cnt - 1), EPW + 2 * BT - 1)
            plsc.store_scatter(cidx, [pos], idxv)
            return cnt + jnp.sum(mi)

        cnt = lax.fori_loop(0, EPW // 16, cmp_body, 0)
        for k in range(2 * BT // 16):
            cidx[pl.ds(cnt + k * 16, 16)] = jnp.full((16,), EPW, jnp.int32)

        def issue(gb, sb, db, eb, hbuf, ebuf, semh, seme):
            for k in range(BT // 16):
                ids = cidx[pl.ds(gb + k * 16, 16)]
                sv = plsc.load_gather(srcw, [ids])
                dv = plsc.load_gather(dstw, [ids])
                lv = dv - lo
                ok = (lv >= 0) & (lv < CH)
                dl = jnp.where(ok, lv, CH)
                sb[pl.ds(k * 16, 16)] = sv
                db[pl.ds(k * 16, 16)] = dl
                eb[pl.ds(k * 16, 16)] = ids + wbase
            pltpu.async_copy(h_hbm.at[sb], hbuf, semh)
            pltpu.async_copy(e_hbm.at[eb], ebuf, seme)

        npair = (cnt + 2 * BT - 1) // (2 * BT)

        def bat(g, carry):
            gb = g * 2 * BT
            issue(gb, sba, dba, eba, hbufa, ebufa, semha, semea)
            issue(gb + BT, sbb, dbb, ebb, hbufb, ebufb, semhb, semeb)
            pltpu.make_async_copy(h_hbm.at[sba], hbufa, semha).wait()
            pltpu.make_async_copy(e_hbm.at[eba], ebufa, semea).wait()
            scale(hbufa, ebufa, msga)
            pltpu.sync_copy(msga, acc_sh.at[dba], add=True)
            pltpu.make_async_copy(h_hbm.at[sbb], hbufb, semhb).wait()
            pltpu.make_async_copy(e_hbm.at[ebb], ebufb, semeb).wait()
            scale(hbufb, ebufb, msgb)
            pltpu.sync_copy(msgb, acc_sh.at[dbb], add=True)
            return carry

        lax.fori_loop(0, npair, bat, 0)
        plsc.subcore_barrier()
        frows = CH // NS
        pltpu.sync_copy(acc_sh.at[pl.ds(s_ * frows, frows)],
                        acc_hbm.at[c, pl.ds(lo + s_ * frows, frows)])
        plsc.subcore_barrier()


def _p2(h, src, dst, e, za):
    bufs = []
    for _ in range(2):
        bufs += [
            pltpu.VMEM((BT,), jnp.int32),
            pltpu.VMEM((BT,), jnp.int32),
            pltpu.VMEM((BT,), jnp.int32),
            pltpu.VMEM((BT, HID), jnp.float32),
            pltpu.VMEM((BT, 16), jnp.float32),
            pltpu.VMEM((BT, HID), jnp.float32),
        ]
    f = pl.kernel(
        _p2_body,
        out_type=jax.ShapeDtypeStruct((NC, NROW, HID), jnp.float32),
        mesh=_mesh(),
        compiler_params=_SC_PARAMS,
        scratch_types=[
            pltpu.VMEM((EPW + 2 * BT,), jnp.int32),
            pltpu.VMEM((EPW + 2 * BT,), jnp.int32),
            pltpu.VMEM((EPW + 2 * BT,), jnp.int32),
        ] + bufs + [
            pltpu.VMEM_SHARED((ACC_R, HID), jnp.float32),
            pltpu.SemaphoreType.DMA,
            pltpu.SemaphoreType.DMA,
            pltpu.SemaphoreType.DMA,
            pltpu.SemaphoreType.DMA,
        ],
    )
    return f(h, src, dst, e, za)


def _pre_body(x_ref, w_ref, b_ref, a_ref, h_ref, al_ref):
    x = x_ref[...]
    h = jnp.dot(x, w_ref[...], preferred_element_type=jnp.float32) + b_ref[...]
    h_ref[...] = h
    al_ref[...] = jnp.dot(h, a_ref[...], preferred_element_type=jnp.float32)


def _pre(x, w, b, acat):
    return pl.pallas_call(
        _pre_body,
        grid=(N // _BLK,),
        in_specs=[
            pl.BlockSpec((_BLK, D_IN), lambda i: (i, 0)),
            pl.BlockSpec((D_IN, HID), lambda i: (0, 0)),
            pl.BlockSpec((1, HID), lambda i: (0, 0)),
            pl.BlockSpec((HID, 32), lambda i: (0, 0)),
        ],
        out_specs=[
            pl.BlockSpec((_BLK, HID), lambda i: (i, 0)),
            pl.BlockSpec((_BLK, 32), lambda i: (i, 0)),
        ],
        out_shape=[
            jax.ShapeDtypeStruct((N, HID), jnp.float32),
            jax.ShapeDtypeStruct((N, 32), jnp.float32),
        ],
    )(x, w, b.reshape(1, HID), acat)


def _post_body(acc1, acc2, s1, s2, kw_ref, kb_ref,
               o1_ref, o2_ref, t_ref):
    @pl.when(pl.program_id(0) == 0)
    def _():
        t_ref[...] = jnp.zeros_like(t_ref)

    row = lax.broadcasted_iota(jnp.int32, (16, HID), 0)
    col = lax.broadcasted_iota(jnp.int32, (16, HID), 1)
    expm = (col // DIM == row).astype(jnp.float32)

    def one(acc, s3):
        s = s3[0] + s3[1]
        r = 1.0 / (s + 1e-16)
        rexp = jnp.dot(r, expm, preferred_element_type=jnp.float32)
        return jax.nn.relu((acc[0] + acc[1]) * rexp)

    o1 = one(acc1, s1)
    o2 = one(acc2, s2)
    o1_ref[...] = o1
    o2_ref[...] = o2
    t1 = jnp.tanh(jnp.dot(o1, kw_ref[...],
                          preferred_element_type=jnp.float32) + kb_ref[...])
    t2 = jnp.tanh(jnp.dot(o2, kw_ref[...],
                          preferred_element_type=jnp.float32) + kb_ref[...])
    t_ref[0, :] += jnp.sum(t1, axis=0)
    t_ref[1, :] += jnp.sum(t2, axis=0)


def _post(acc1, acc2, s1, s2, k_W, k_b):
    bspec_a = pl.BlockSpec((NC, _BLK, HID), lambda i: (0, i, 0))
    bspec_s = pl.BlockSpec((NC, _BLK, 16), lambda i: (0, i, 0))
    return pl.pallas_call(
        _post_body,
        grid=(N // _BLK,),
        in_specs=[bspec_a, bspec_a, bspec_s, bspec_s,
                  pl.BlockSpec((HID, HID), lambda i: (0, 0)),
                  pl.BlockSpec((1, HID), lambda i: (0, 0))],
        out_specs=[pl.BlockSpec((_BLK, HID), lambda i: (i, 0)),
                   pl.BlockSpec((_BLK, HID), lambda i: (i, 0)),
                   pl.BlockSpec((2, HID), lambda i: (0, 0))],
        out_shape=[jax.ShapeDtypeStruct((N, HID), jnp.float32),
                   jax.ShapeDtypeStruct((N, HID), jnp.float32),
                   jax.ShapeDtypeStruct((2, HID), jnp.float32)],
    )(acc1, acc2, s1, s2, k_W, k_b.reshape(1, HID))


def _final_body(o1_ref, o2_ref, w_ref, b_ref, beta_ref, out_ref):
    mix = beta_ref[0, 0] * o1_ref[...] + beta_ref[0, 1] * o2_ref[...]
    out_ref[...] = jnp.dot(mix, w_ref[...],
                           preferred_element_type=jnp.float32) + b_ref[...]


def _final(o1, o2, lin_W, lin_b, beta):
    return pl.pallas_call(
        _final_body,
        grid=(N // _BLK,),
        in_specs=[
            pl.BlockSpec((_BLK, HID), lambda i: (i, 0)),
            pl.BlockSpec((_BLK, HID), lambda i: (i, 0)),
            pl.BlockSpec((HID, OUT), lambda i: (0, 0)),
            pl.BlockSpec((1, OUT), lambda i: (0, 0)),
            pl.BlockSpec(memory_space=pltpu.SMEM),
        ],
        out_specs=pl.BlockSpec((_BLK, OUT), lambda i: (i, 0)),
        out_shape=jax.ShapeDtypeStruct((N, OUT), jnp.float32),
    )(o1, o2, lin_W, lin_b.reshape(1, OUT), beta.reshape(1, 2))


def kernel(x_litter, edge_index_rel1, edge_index_rel2, proj_W, proj_b,
           att_src_1, att_dst_1, att_src_2, att_dst_2,
           k_W, k_b, q, lin_W, lin_b):
    def expand(att):
        eye = jnp.eye(HEADS, dtype=jnp.float32)
        return (att[:, :, None] * eye[:, None, :]).reshape(HID, HEADS)

    acat = jnp.concatenate([expand(att_src_1), expand(att_dst_1),
                            expand(att_src_2), expand(att_dst_2)], axis=1)

    h, al = _pre(x_litter, proj_W, proj_b, acat)

    def dup(a8):
        return jnp.pad(jnp.concatenate([a8, a8], axis=1),
                       ((0, NS_TAB - N), (0, 0)))

    asd1 = dup(al[:, 0:8])
    add1 = dup(al[:, 8:16])
    asd2 = dup(al[:, 16:24])
    add2 = dup(al[:, 24:32])

    def padded(ei):
        src = jnp.pad(ei[0], (0, NE_PAD - E))
        dst = jnp.pad(ei[1], (0, NE_PAD - E), constant_values=N)
        return src, dst

    src1, dst1 = padded(edge_index_rel1)
    src2, dst2 = padded(edge_index_rel2)

    zs = jnp.zeros((NS_TAB // NS, 16), jnp.float32)
    za = jnp.zeros((ACC_R // NS, HID), jnp.float32)

    e1, s1 = _p1(asd1, add1, src1, dst1, zs)
    e2, s2 = _p1(asd2, add2, src2, dst2, zs)
    acc1 = _p2(h, src1, dst1, e1, za)
    acc2 = _p2(h, src2, dst2, e2, za)

    o1, o2, tsum = _post(acc1, acc2, s1, s2, k_W, k_b)
    score = (q[None, :] * (tsum / N)).sum(-1)
    beta = jax.nn.softmax(score, axis=0)
    return _final(o1, o2, lin_W, lin_b, beta)
